# baseline (speedup 1.0000x reference)
import functools

import jax
import jax.numpy as jnp
from jax import lax
from jax.experimental import pallas as pl
from jax.experimental.pallas import tpu as pltpu
from jax.experimental.pallas import tpu_sc as plsc

_S = 30.0
_BLOCK = 1024


def _sc_gather(B, C):
    info = plsc.get_sparse_core_info()
    NC, NS, L = info.num_cores, info.num_subcores, info.num_lanes
    NW = NC * NS
    bw = B // NW
    KC = bw // 128
    mesh = plsc.VectorSubcoreMesh(core_axis_name="c", subcore_axis_name="s")

    @functools.partial(
        pl.kernel,
        mesh=mesh,
        out_type=[
            jax.ShapeDtypeStruct((B,), jnp.float32),
            jax.ShapeDtypeStruct((B,), jnp.float32),
        ],
        scratch_types=[
            pltpu.VMEM((bw,), jnp.int32),
            pltpu.VMEM((bw,), jnp.int32),
            pltpu.VMEM((bw,), jnp.float32),
            pltpu.VMEM((bw,), jnp.float32),
            pltpu.VMEM((bw,), jnp.float32),
            pltpu.SemaphoreType.DMA,
        ],
    )
    def k(logits_hbm, m_hbm, tgt_hbm, xt_out, adj_out,
          tgt_v, idx_v, xt_v, mt_v, adj_v, sem):
        wid = lax.axis_index("s") * NC + lax.axis_index("c")
        base = wid * bw
        pltpu.sync_copy(tgt_hbm.at[pl.ds(base, bw)], tgt_v)
        lanes = lax.iota(jnp.int32, L)
        for j in range(bw // L):
            t16 = tgt_v[pl.ds(j * L, L)]
            rows = (base + j * L) + lanes
            idx_v[pl.ds(j * L, L)] = rows * C + t16
        copies = [
            pltpu.async_copy(logits_hbm.at[idx_v.at[pl.ds(kk * 128, 128)]],
                             xt_v.at[pl.ds(kk * 128, 128)], sem)
            for kk in range(KC)
        ] + [
            pltpu.async_copy(m_hbm.at[tgt_v.at[pl.ds(kk * 128, 128)]],
                             mt_v.at[pl.ds(kk * 128, 128)], sem)
            for kk in range(KC)
        ]
        for cp in copies:
            cp.wait()
        for j in range(bw // L):
            adj_v[pl.ds(j * L, L)] = (xt_v[pl.ds(j * L, L)]
                                      - _S * mt_v[pl.ds(j * L, L)])
        pltpu.sync_copy(xt_v, xt_out.at[pl.ds(base, bw)])
        pltpu.sync_copy(adj_v, adj_out.at[pl.ds(base, bw)])

    return k


def _ce_kernel(logits_ref, xt_ref, adj_ref, out_ref):
    i = pl.program_id(0)
    x = logits_ref[...]
    xt = xt_ref[...]
    adj = adj_ref[...]
    mx = jnp.max(x, axis=1, keepdims=True)
    z = jnp.sum(jnp.exp(x - mx), axis=1, keepdims=True)
    zadj = z - jnp.exp(xt - mx) + jnp.exp(adj - mx)
    part = jnp.sum(mx + jnp.log(zadj) - adj)

    @pl.when(i == 0)
    def _():
        out_ref[...] = jnp.zeros_like(out_ref)

    out_ref[...] += jnp.full((1, 1), part, jnp.float32)


def kernel(logits, m_list, target):
    B, C = logits.shape
    xt, adj = _sc_gather(B, C)(logits.reshape(-1), m_list, target)
    out = pl.pallas_call(
        _ce_kernel,
        grid=(B // _BLOCK,),
        in_specs=[
            pl.BlockSpec((_BLOCK, C), lambda i: (i, 0)),
            pl.BlockSpec((_BLOCK, 1), lambda i: (i, 0)),
            pl.BlockSpec((_BLOCK, 1), lambda i: (i, 0)),
        ],
        out_specs=pl.BlockSpec((1, 1), lambda i: (0, 0)),
        out_shape=jax.ShapeDtypeStruct((1, 1), jnp.float32),
    )(logits, xt.reshape(B, 1), adj.reshape(B, 1))
    return (out[0, 0] / B).astype(jnp.float32)


# speedup vs baseline: 1.1924x; 1.1924x over previous
import functools

import jax
import jax.numpy as jnp
from jax import lax
from jax.experimental import pallas as pl
from jax.experimental.pallas import tpu as pltpu
from jax.experimental.pallas import tpu_sc as plsc

_S = 30.0
_BLOCK = 4096


def _sc_margin_gather(B):
    info = plsc.get_sparse_core_info()
    NC, NS, L = info.num_cores, info.num_subcores, info.num_lanes
    NW = NC * NS
    bw = B // NW
    KC = bw // 128
    mesh = plsc.VectorSubcoreMesh(core_axis_name="c", subcore_axis_name="s")

    @functools.partial(
        pl.kernel,
        mesh=mesh,
        out_type=jax.ShapeDtypeStruct((B,), jnp.float32),
        scratch_types=[
            pltpu.VMEM((bw,), jnp.int32),
            pltpu.VMEM((bw,), jnp.float32),
            pltpu.SemaphoreType.DMA,
        ],
    )
    def k(m_hbm, tgt_hbm, mt_out, tgt_v, mt_v, sem):
        wid = lax.axis_index("s") * NC + lax.axis_index("c")
        base = wid * bw
        pltpu.sync_copy(tgt_hbm.at[pl.ds(base, bw)], tgt_v)
        copies = [
            pltpu.async_copy(m_hbm.at[tgt_v.at[pl.ds(kk * 128, 128)]],
                             mt_v.at[pl.ds(kk * 128, 128)], sem)
            for kk in range(KC)
        ]
        for cp in copies:
            cp.wait()
        pltpu.sync_copy(mt_v, mt_out.at[pl.ds(base, bw)])

    return k


def _ce_kernel(logits_ref, mt_ref, tgt_ref, out_ref):
    i = pl.program_id(0)
    x = logits_ref[...]
    mt = mt_ref[...]
    t = tgt_ref[...]
    col = lax.broadcasted_iota(jnp.int32, x.shape, 1)
    onehot = col == t
    adj = x - jnp.where(onehot, _S * mt, 0.0)
    mx = jnp.max(adj, axis=1, keepdims=True)
    z = jnp.sum(jnp.exp(adj - mx), axis=1, keepdims=True)
    xt = jnp.sum(jnp.where(onehot, adj, 0.0), axis=1, keepdims=True)
    part = jnp.sum(mx + jnp.log(z) - xt)

    @pl.when(i == 0)
    def _():
        out_ref[...] = jnp.zeros_like(out_ref)

    out_ref[...] += jnp.full((1, 1), part, jnp.float32)


def kernel(logits, m_list, target):
    B, C = logits.shape
    mt = _sc_margin_gather(B)(m_list, target)
    out = pl.pallas_call(
        _ce_kernel,
        grid=(B // _BLOCK,),
        in_specs=[
            pl.BlockSpec((_BLOCK, C), lambda i: (i, 0)),
            pl.BlockSpec((_BLOCK, 1), lambda i: (i, 0)),
            pl.BlockSpec((_BLOCK, 1), lambda i: (i, 0)),
        ],
        out_specs=pl.BlockSpec((1, 1), lambda i: (0, 0)),
        out_shape=jax.ShapeDtypeStruct((1, 1), jnp.float32),
    )(logits, mt.reshape(B, 1), target.reshape(B, 1))
    return (out[0, 0] / B).astype(jnp.float32)
